# baseline (speedup 1.0000x reference)
import jax
import jax.numpy as jnp
from jax import lax
from jax.experimental import pallas as pl
from jax.experimental.pallas import tpu as pltpu
from jax.experimental.pallas import tpu_sc as plsc

TOP_K = 16
L = 16
HW = 1024
NCHUNK = HW // L
NC = 2
NS = 16
NW = NC * NS
ROWS_G = 16


def _sc_body(num_rows, num_ch, x_hbm, w_hbm, o_hbm, xbuf, wbuf, cand, obuf):
    rpw = num_rows // NW
    ngroups = rpw // ROWS_G
    wid = lax.axis_index("s") * NC + lax.axis_index("c")
    base = wid * rpw

    pltpu.sync_copy(w_hbm, wbuf)
    lane = lax.iota(jnp.int32, L)
    neg = jnp.full((L,), -jnp.inf, jnp.float32)

    def group_body(g, carry):
        pltpu.sync_copy(
            x_hbm.at[pl.ds((base + g * ROWS_G) * HW, ROWS_G * HW)], xbuf)

        def row_body(r, ovec):
            roff = r * HW
            m = xbuf[pl.ds(roff, L)]
            for k in range(1, NCHUNK):
                m = jnp.maximum(m, xbuf[pl.ds(roff + k * L, L)])
            ms, _ = plsc.sort_key_val(m, m, descending=False)
            tv = jnp.broadcast_to(ms[0], (L,))
            off = jnp.int32(0)
            for k in range(NCHUNK):
                v = xbuf[pl.ds(roff + k * L, L)]
                msk = v >= tv
                pos = plsc.cumsum(jnp.where(msk, jnp.int32(1), jnp.int32(0)))
                plsc.store_scatter(cand, [pos + (off - 1)], v, mask=msk)
                off = off + pos[L - 1]
            plsc.store_scatter(cand, [lane + off], neg)
            k0 = cand[pl.ds(0, L)]
            top, _ = plsc.sort_key_val(k0, k0, descending=True)
            nv = (off + jnp.int32(15)) >> 4

            def merge(j, top):
                v = plsc.load_gather(cand, [lane + j * L])
                va, _ = plsc.sort_key_val(v, v, descending=False)
                mx = jnp.maximum(top, va)
                t2, _ = plsc.sort_key_val(mx, mx, descending=True)
                return t2

            top = lax.fori_loop(1, nv, merge, top)
            ch = lax.rem(base + g * ROWS_G + r, jnp.int32(num_ch))
            wv = plsc.load_gather(wbuf, [lane + ch * TOP_K])
            s = plsc.cumsum(top * wv)[L - 1]
            return jnp.where(lane == r, jnp.broadcast_to(s, (L,)), ovec)

        ovec = lax.fori_loop(0, ROWS_G, row_body, jnp.zeros((L,), jnp.float32))
        plsc.store_scatter(obuf, [lane + g * L], ovec)
        return carry

    lax.fori_loop(0, ngroups, group_body, jnp.int32(0))
    pltpu.sync_copy(obuf, o_hbm.at[pl.ds(base, rpw)])


def kernel(x, weights):
    B, C = x.shape[0], x.shape[1]
    num_rows = B * C
    x_flat = x.reshape(num_rows * HW)
    w_flat = weights.reshape(-1)
    rpw = num_rows // NW

    body = lambda *refs: _sc_body(num_rows, C, *refs)
    out = pl.kernel(
        body,
        out_type=jax.ShapeDtypeStruct((num_rows,), jnp.float32),
        mesh=plsc.VectorSubcoreMesh(core_axis_name="c", subcore_axis_name="s"),
        compiler_params=pltpu.CompilerParams(needs_layout_passes=False),
        scratch_types=[
            pltpu.VMEM((ROWS_G * HW,), jnp.float32),
            pltpu.VMEM((C * TOP_K,), jnp.float32),
            pltpu.VMEM((HW + 4 * L,), jnp.float32),
            pltpu.VMEM((rpw,), jnp.float32),
        ],
    )(x_flat, w_flat)
    return out.reshape(B, C)


# speedup vs baseline: 1.2381x; 1.2381x over previous
import jax
import jax.numpy as jnp
from jax import lax
from jax.experimental import pallas as pl
from jax.experimental.pallas import tpu as pltpu
from jax.experimental.pallas import tpu_sc as plsc

TOP_K = 16
L = 16
HW = 1024
NGRP = HW // L
NC = 2
NS = 16
NW = NC * NS
ROWS_G = 16


def _batcher_pairs(n):
    pairs = []

    def merge(lo, m, r):
        step = r * 2
        if step < m:
            merge(lo, m, step)
            merge(lo + r, m, step)
            pairs.extend((i, i + r) for i in range(lo + r, lo + m - r, step))
        else:
            pairs.append((lo, lo + r))

    def net(lo, m):
        if m > 1:
            h = m // 2
            net(lo, h)
            net(lo + h, h)
            merge(lo, m, 1)

    net(0, n)
    return pairs


_SORT16 = _batcher_pairs(16)


def _sc_body(num_rows, num_ch, x_hbm, w_hbm, o_hbm, xbuf, wbuf, obuf):
    rpw = num_rows // NW
    nblocks = rpw // ROWS_G
    wid = lax.axis_index("s") * NC + lax.axis_index("c")
    base = wid * rpw

    pltpu.sync_copy(w_hbm, wbuf)
    lane = lax.iota(jnp.int32, L)
    lanebase = lane * HW
    neg_inf = jnp.full((L,), -jnp.inf, jnp.float32)

    def block_body(g, carry):
        pltpu.sync_copy(
            x_hbm.at[pl.ds((base + g * ROWS_G) * HW, ROWS_G * HW)], xbuf)

        def group_body(j, T):
            idx0 = lanebase + j * L
            V = [plsc.load_gather(xbuf, [idx0 + k]) for k in range(L)]
            for a, b in _SORT16:
                lo = jnp.minimum(V[a], V[b])
                hi = jnp.maximum(V[a], V[b])
                V[a], V[b] = lo, hi
            z = [jnp.maximum(T[i], V[i]) for i in range(L)]
            for d in (8, 4, 2, 1):
                for i in range(L):
                    if i & d == 0:
                        hi = jnp.maximum(z[i], z[i + d])
                        lo = jnp.minimum(z[i], z[i + d])
                        z[i], z[i + d] = hi, lo
            return tuple(z)

        T = lax.fori_loop(0, NGRP, group_body, (neg_inf,) * L)

        ch = lax.rem(g * ROWS_G + lane, jnp.int32(num_ch))
        widx = ch * TOP_K
        acc = T[0] * plsc.load_gather(wbuf, [widx])
        for i in range(1, TOP_K):
            acc = acc + T[i] * plsc.load_gather(wbuf, [widx + i])
        plsc.store_scatter(obuf, [g * L + lane], acc)
        return carry

    lax.fori_loop(0, nblocks, block_body, jnp.int32(0))
    pltpu.sync_copy(obuf, o_hbm.at[pl.ds(base, rpw)])


def kernel(x, weights):
    B, C = x.shape[0], x.shape[1]
    num_rows = B * C
    x_flat = x.reshape(num_rows * HW)
    w_flat = weights.reshape(-1)
    rpw = num_rows // NW

    body = lambda *refs: _sc_body(num_rows, C, *refs)
    out = pl.kernel(
        body,
        out_type=jax.ShapeDtypeStruct((num_rows,), jnp.float32),
        mesh=plsc.VectorSubcoreMesh(core_axis_name="c", subcore_axis_name="s"),
        compiler_params=pltpu.CompilerParams(needs_layout_passes=False),
        scratch_types=[
            pltpu.VMEM((ROWS_G * HW,), jnp.float32),
            pltpu.VMEM((C * TOP_K,), jnp.float32),
            pltpu.VMEM((rpw,), jnp.float32),
        ],
    )(x_flat, w_flat)
    return out.reshape(B, C)


# speedup vs baseline: 1.7895x; 1.4453x over previous
import jax
import jax.numpy as jnp
from jax import lax
from jax.experimental import pallas as pl
from jax.experimental.pallas import tpu as pltpu
from jax.experimental.pallas import tpu_sc as plsc

TOP_K = 16
L = 16
HW = 1024
NGRP = HW // L
NC = 2
NS = 16
NW = NC * NS
ROWS_G = 16
STRIDE = HW + 8


def _batcher_pairs(n):
    pairs = []

    def merge(lo, m, r):
        step = r * 2
        if step < m:
            merge(lo, m, step)
            merge(lo + r, m, step)
            pairs.extend((i, i + r) for i in range(lo + r, lo + m - r, step))
        else:
            pairs.append((lo, lo + r))

    def net(lo, m):
        if m > 1:
            h = m // 2
            net(lo, h)
            net(lo + h, h)
            merge(lo, m, 1)

    net(0, n)
    return pairs


_SORT16 = _batcher_pairs(16)


def _sc_body(num_rows, num_ch, x_hbm, w_hbm, o_hbm, xbuf, wbuf, obuf, sem):
    rpw = num_rows // NW
    nblocks = rpw // ROWS_G
    wid = lax.axis_index("s") * NC + lax.axis_index("c")
    base = wid * rpw

    pltpu.sync_copy(w_hbm, wbuf)
    lane = lax.iota(jnp.int32, L)
    lanebase = lane * STRIDE
    neg_inf = jnp.full((L,), -jnp.inf, jnp.float32)

    def block_body(g, carry):
        row0 = (base + g * ROWS_G) * HW
        copies = [
            pltpu.async_copy(
                x_hbm.at[pl.ds(row0 + r * HW, HW)],
                xbuf.at[pl.ds(r * STRIDE, HW)],
                sem,
            )
            for r in range(ROWS_G)
        ]
        for c in copies:
            c.wait()

        def group_body(j, T):
            idx0 = lanebase + j * L
            V = [plsc.load_gather(xbuf, [idx0 + k]) for k in range(L)]
            for a, b in _SORT16:
                lo = jnp.minimum(V[a], V[b])
                hi = jnp.maximum(V[a], V[b])
                V[a], V[b] = lo, hi
            z = [jnp.maximum(T[i], V[i]) for i in range(L)]
            for d in (8, 4, 2, 1):
                for i in range(L):
                    if i & d == 0:
                        hi = jnp.maximum(z[i], z[i + d])
                        lo = jnp.minimum(z[i], z[i + d])
                        z[i], z[i + d] = hi, lo
            return tuple(z)

        T = lax.fori_loop(0, NGRP, group_body, (neg_inf,) * L)

        ch = lax.rem(g * ROWS_G + lane, jnp.int32(num_ch))
        widx = ch * TOP_K
        acc = T[0] * plsc.load_gather(wbuf, [widx])
        for i in range(1, TOP_K):
            acc = acc + T[i] * plsc.load_gather(wbuf, [widx + i])
        plsc.store_scatter(obuf, [g * L + lane], acc)
        return carry

    lax.fori_loop(0, nblocks, block_body, jnp.int32(0))
    pltpu.sync_copy(obuf, o_hbm.at[pl.ds(base, rpw)])


def kernel(x, weights):
    B, C = x.shape[0], x.shape[1]
    num_rows = B * C
    x_flat = x.reshape(num_rows * HW)
    w_flat = weights.reshape(-1)
    rpw = num_rows // NW

    body = lambda *refs: _sc_body(num_rows, C, *refs)
    out = pl.kernel(
        body,
        out_type=jax.ShapeDtypeStruct((num_rows,), jnp.float32),
        mesh=plsc.VectorSubcoreMesh(core_axis_name="c", subcore_axis_name="s"),
        compiler_params=pltpu.CompilerParams(needs_layout_passes=False),
        scratch_types=[
            pltpu.VMEM((ROWS_G * STRIDE,), jnp.float32),
            pltpu.VMEM((C * TOP_K,), jnp.float32),
            pltpu.VMEM((rpw,), jnp.float32),
            pltpu.SemaphoreType.DMA,
        ],
    )(x_flat, w_flat)
    return out.reshape(B, C)


# speedup vs baseline: 1.8032x; 1.0076x over previous
import jax
import jax.numpy as jnp
from jax import lax
from jax.experimental import pallas as pl
from jax.experimental.pallas import tpu as pltpu
from jax.experimental.pallas import tpu_sc as plsc

TOP_K = 16
L = 16
HW = 1024
NGRP = HW // L
NC = 2
NS = 16
NW = NC * NS
ROWS_G = 16


def _batcher_pairs(n):
    pairs = []

    def merge(lo, m, r):
        step = r * 2
        if step < m:
            merge(lo, m, step)
            merge(lo + r, m, step)
            pairs.extend((i, i + r) for i in range(lo + r, lo + m - r, step))
        else:
            pairs.append((lo, lo + r))

    def net(lo, m):
        if m > 1:
            h = m // 2
            net(lo, h)
            net(lo + h, h)
            merge(lo, m, 1)

    net(0, n)
    return pairs


_SORT16 = _batcher_pairs(16)


def _sc_body(num_rows, num_ch, x_hbm, w_hbm, o_hbm, xbuf, wbuf, obuf, sem):
    rpw = num_rows // NW
    nblocks = rpw // ROWS_G
    wid = lax.axis_index("s") * NC + lax.axis_index("c")
    base = wid * rpw

    pltpu.sync_copy(w_hbm, wbuf)
    lane = lax.iota(jnp.int32, L)
    lanebase = lane * HW
    neg_inf = jnp.full((L,), -jnp.inf, jnp.float32)

    def block_body(g, carry):
        pltpu.sync_copy(
            x_hbm.at[pl.ds((base + g * ROWS_G) * HW, ROWS_G * HW)], xbuf)

        def group_body(j, T):
            idx0 = lanebase + j * L
            V = [plsc.load_gather(xbuf, [idx0 + (lane ^ k)]) for k in range(L)]
            for a, b in _SORT16:
                lo = jnp.minimum(V[a], V[b])
                hi = jnp.maximum(V[a], V[b])
                V[a], V[b] = lo, hi
            z = [jnp.maximum(T[i], V[i]) for i in range(L)]
            for d in (8, 4, 2, 1):
                for i in range(L):
                    if i & d == 0:
                        hi = jnp.maximum(z[i], z[i + d])
                        lo = jnp.minimum(z[i], z[i + d])
                        z[i], z[i + d] = hi, lo
            return tuple(z)

        T = lax.fori_loop(0, NGRP, group_body, (neg_inf,) * L)

        ch = lax.rem(g * ROWS_G + lane, jnp.int32(num_ch))
        widx = ch * TOP_K
        acc = T[0] * plsc.load_gather(wbuf, [widx])
        for i in range(1, TOP_K):
            acc = acc + T[i] * plsc.load_gather(wbuf, [widx + i])
        plsc.store_scatter(obuf, [g * L + lane], acc)
        return carry

    lax.fori_loop(0, nblocks, block_body, jnp.int32(0))
    pltpu.sync_copy(obuf, o_hbm.at[pl.ds(base, rpw)])


def kernel(x, weights):
    B, C = x.shape[0], x.shape[1]
    num_rows = B * C
    x_flat = x.reshape(num_rows * HW)
    w_flat = weights.reshape(-1)
    rpw = num_rows // NW

    body = lambda *refs: _sc_body(num_rows, C, *refs)
    out = pl.kernel(
        body,
        out_type=jax.ShapeDtypeStruct((num_rows,), jnp.float32),
        mesh=plsc.VectorSubcoreMesh(core_axis_name="c", subcore_axis_name="s"),
        compiler_params=pltpu.CompilerParams(needs_layout_passes=False),
        scratch_types=[
            pltpu.VMEM((ROWS_G * HW,), jnp.float32),
            pltpu.VMEM((C * TOP_K,), jnp.float32),
            pltpu.VMEM((rpw,), jnp.float32),
            pltpu.SemaphoreType.DMA,
        ],
    )(x_flat, w_flat)
    return out.reshape(B, C)


# speedup vs baseline: 1.8542x; 1.0283x over previous
import jax
import jax.numpy as jnp
from jax import lax
from jax.experimental import pallas as pl
from jax.experimental.pallas import tpu as pltpu
from jax.experimental.pallas import tpu_sc as plsc

TOP_K = 16
L = 16
HW = 1024
NGRP = HW // L
NC = 2
NS = 16
NW = NC * NS
ROWS_G = 16


def _batcher_pairs(n):
    pairs = []

    def merge(lo, m, r):
        step = r * 2
        if step < m:
            merge(lo, m, step)
            merge(lo + r, m, step)
            pairs.extend((i, i + r) for i in range(lo + r, lo + m - r, step))
        else:
            pairs.append((lo, lo + r))

    def net(lo, m):
        if m > 1:
            h = m // 2
            net(lo, h)
            net(lo + h, h)
            merge(lo, m, 1)

    net(0, n)
    return pairs


_SORT8 = _batcher_pairs(8)


def _sc_body(num_rows, num_ch, x_hbm, w_hbm, o_hbm, xbuf, wbuf, obuf, sem):
    rpw = num_rows // NW
    nblocks = rpw // ROWS_G
    wid = lax.axis_index("s") * NC + lax.axis_index("c")
    base = wid * rpw

    pltpu.sync_copy(w_hbm, wbuf)
    lane = lax.iota(jnp.int32, L)
    lanebase = lane * HW
    neg_inf = jnp.full((L,), -jnp.inf, jnp.float32)

    def block_body(g, carry):
        pltpu.sync_copy(
            x_hbm.at[pl.ds((base + g * ROWS_G) * HW, ROWS_G * HW)], xbuf)

        def group_body(j, T):
            idx0 = lanebase + j * 8
            V = [plsc.load_gather(xbuf, [idx0 + (lane ^ k)]) for k in range(8)]
            for a, b in _SORT8:
                lo = jnp.minimum(V[a], V[b])
                hi = jnp.maximum(V[a], V[b])
                V[a], V[b] = lo, hi
            z = list(T)
            for i in range(8):
                z[8 + i] = jnp.maximum(T[8 + i], V[i])
            for d in (8, 4, 2, 1):
                for i in range(L):
                    if i & d == 0:
                        hi = jnp.maximum(z[i], z[i + d])
                        lo = jnp.minimum(z[i], z[i + d])
                        z[i], z[i + d] = hi, lo
            return tuple(z)

        T = lax.fori_loop(0, HW // 8, group_body, (neg_inf,) * L)

        ch = lax.rem(g * ROWS_G + lane, jnp.int32(num_ch))
        widx = ch * TOP_K
        acc = T[0] * plsc.load_gather(wbuf, [widx])
        for i in range(1, TOP_K):
            acc = acc + T[i] * plsc.load_gather(wbuf, [widx + i])
        plsc.store_scatter(obuf, [g * L + lane], acc)
        return carry

    lax.fori_loop(0, nblocks, block_body, jnp.int32(0))
    pltpu.sync_copy(obuf, o_hbm.at[pl.ds(base, rpw)])


def kernel(x, weights):
    B, C = x.shape[0], x.shape[1]
    num_rows = B * C
    x_flat = x.reshape(num_rows * HW)
    w_flat = weights.reshape(-1)
    rpw = num_rows // NW

    body = lambda *refs: _sc_body(num_rows, C, *refs)
    out = pl.kernel(
        body,
        out_type=jax.ShapeDtypeStruct((num_rows,), jnp.float32),
        mesh=plsc.VectorSubcoreMesh(core_axis_name="c", subcore_axis_name="s"),
        compiler_params=pltpu.CompilerParams(needs_layout_passes=False),
        scratch_types=[
            pltpu.VMEM((ROWS_G * HW,), jnp.float32),
            pltpu.VMEM((C * TOP_K,), jnp.float32),
            pltpu.VMEM((rpw,), jnp.float32),
            pltpu.SemaphoreType.DMA,
        ],
    )(x_flat, w_flat)
    return out.reshape(B, C)
